# baseline (speedup 1.0000x reference)
import functools

import numpy as np
import jax
import jax.numpy as jnp
from jax import lax
from jax.experimental import pallas as pl
from jax.experimental.pallas import tpu as pltpu
from jax.experimental.pallas import tpu_sc as plsc

N = 10000
NP = 10240
D = 128
NH = 8
DH = 16
E = 320000
NC = 2
NS = 16
NW = NC * NS
EPER = E // NW
EPT = 10240
C = 64
G = 16
NG = C // G
NR = 10
RANGE = NP // NR
RT = 1152
RSTRIPE = RT // NS
DRW = RANGE + 128
SCC = 320
NSC = EPT // SCC
STRIPE = NP // NS
BLK = 1024
GRID = NP // BLK

_SEL = np.zeros((D, NH), np.float32)
for _d in range(D):
    _SEL[_d, _d // DH] = 1.0
_REP = np.zeros((2 * NH, D), np.float32)
for _h in range(NH):
    _REP[_h, _h * DH:(_h + 1) * DH] = 1.0
_REP8 = np.zeros((NH, D), np.float32)
for _h in range(NH):
    _REP8[_h, _h * DH:(_h + 1) * DH] = 1.0


def _proj_body(x_ref, w_ref, b_ref, o_ref):
    o_ref[...] = jnp.dot(x_ref[...], w_ref[...],
                         preferred_element_type=jnp.float32) + b_ref[...]


def _proj(x, w, b):
    k = x.shape[1]
    return pl.pallas_call(
        _proj_body,
        grid=(GRID,),
        in_specs=[pl.BlockSpec((BLK, k), lambda i: (i, 0)),
                  pl.BlockSpec((k, D), lambda i: (0, 0)),
                  pl.BlockSpec((1, D), lambda i: (0, 0))],
        out_specs=pl.BlockSpec((BLK, D), lambda i: (i, 0)),
        out_shape=jax.ShapeDtypeStruct((NP, D), jnp.float32),
    )(x, w, b)


def _pre_body(h_ref, w_ref, as_ref, ad_ref, sel_ref, rep_ref,
              ho_ref, ta_ref, tb_ref, so_ref, ds_ref):
    h = jnp.dot(h_ref[...], w_ref[...], preferred_element_type=jnp.float32)
    ho_ref[...] = h
    sel = sel_ref[...]
    als = jnp.dot(h * as_ref[...], sel, preferred_element_type=jnp.float32)
    ald = jnp.dot(h * ad_ref[...], sel, preferred_element_type=jnp.float32)
    ta_ref[...] = jnp.tile(als, (1, D // NH))
    tb_ref[...] = jnp.tile(ald, (1, D // NH))
    es = als + ald
    exs = jnp.exp(jnp.where(es >= 0.0, es, 0.2 * es))
    ds2 = jnp.concatenate([exs, exs], axis=1)
    ds_ref[...] = ds2
    so_ref[...] = h * jnp.dot(ds2, rep_ref[...],
                              preferred_element_type=jnp.float32)


def _pre(h, w, asf, adf):
    return pl.pallas_call(
        _pre_body,
        grid=(GRID,),
        in_specs=[pl.BlockSpec((BLK, D), lambda i: (i, 0)),
                  pl.BlockSpec((D, D), lambda i: (0, 0)),
                  pl.BlockSpec((1, D), lambda i: (0, 0)),
                  pl.BlockSpec((1, D), lambda i: (0, 0)),
                  pl.BlockSpec((D, NH), lambda i: (0, 0)),
                  pl.BlockSpec((2 * NH, D), lambda i: (0, 0))],
        out_specs=[pl.BlockSpec((BLK, D), lambda i: (i, 0)),
                   pl.BlockSpec((BLK, D), lambda i: (i, 0)),
                   pl.BlockSpec((BLK, D), lambda i: (i, 0)),
                   pl.BlockSpec((BLK, D), lambda i: (i, 0)),
                   pl.BlockSpec((BLK, 2 * NH), lambda i: (i, 0))],
        out_shape=[jax.ShapeDtypeStruct((NP, D), jnp.float32),
                   jax.ShapeDtypeStruct((NP, D), jnp.float32),
                   jax.ShapeDtypeStruct((NP, D), jnp.float32),
                   jax.ShapeDtypeStruct((NP, D), jnp.float32),
                   jax.ShapeDtypeStruct((NP, 2 * NH), jnp.float32)],
    )(h, w, asf, adf, _SEL, _REP)


def _post_body(o0_ref, o1_ref, so_ref, dsc_ref, ds_ref, b_ref, rep_ref,
               rep8_ref, ho_ref):
    dsum = jnp.sum(dsc_ref[...], axis=0)
    seg = lax.dot_general(dsum, rep8_ref[...], (((0,), (0,)), ((), ())),
                          preferred_element_type=jnp.float32)
    den = seg + jnp.dot(ds_ref[...], rep_ref[...],
                        preferred_element_type=jnp.float32)
    val = (o0_ref[...] + o1_ref[...] + so_ref[...]) / (den + 1e-16) + b_ref[...]
    act = jnp.where(val > 0.0, val, jnp.exp(val) - 1.0)
    rows = pl.program_id(0) * BLK + lax.broadcasted_iota(jnp.int32, (BLK, 1), 0)
    ho_ref[...] = jnp.where(rows < N, act, 0.0)


def _post(o0, o1, so, dsc, ds, b):
    return pl.pallas_call(
        _post_body,
        grid=(GRID,),
        in_specs=[pl.BlockSpec((BLK, D), lambda i: (i, 0)),
                  pl.BlockSpec((BLK, D), lambda i: (i, 0)),
                  pl.BlockSpec((BLK, D), lambda i: (i, 0)),
                  pl.BlockSpec((NW, NH, BLK), lambda i: (0, 0, i)),
                  pl.BlockSpec((BLK, 2 * NH), lambda i: (i, 0)),
                  pl.BlockSpec((1, D), lambda i: (0, 0)),
                  pl.BlockSpec((2 * NH, D), lambda i: (0, 0)),
                  pl.BlockSpec((NH, D), lambda i: (0, 0))],
        out_specs=pl.BlockSpec((BLK, D), lambda i: (i, 0)),
        out_shape=jax.ShapeDtypeStruct((NP, D), jnp.float32),
    )(o0, o1, so, dsc, ds, b, _REP, _REP8)


def _sem_body(e0_ref, e1_ref, e2_ref, w1_ref, b1_ref, w2_ref,
              wc1_ref, bc1_ref, wc2_ref, bc2_ref, o_ref):
    embs = [e0_ref[...], e1_ref[...], e2_ref[...]]
    w1 = w1_ref[...]
    b1 = b1_ref[...]
    w2 = w2_ref[...]
    ss = []
    for e in embs:
        t = jnp.tanh(jnp.dot(e, w1, preferred_element_type=jnp.float32) + b1)
        ss.append(jnp.sum(t * w2, axis=1, keepdims=True))
    m = jnp.maximum(jnp.maximum(ss[0], ss[1]), ss[2])
    ws = [jnp.exp(s - m) for s in ss]
    tot = ws[0] + ws[1] + ws[2]
    z = (ws[0] * embs[0] + ws[1] * embs[1] + ws[2] * embs[2]) / tot
    t1 = jnp.maximum(jnp.dot(z, wc1_ref[...],
                             preferred_element_type=jnp.float32) + bc1_ref[...],
                     0.0)
    o_ref[...] = jnp.dot(t1, wc2_ref[...],
                         preferred_element_type=jnp.float32) + bc2_ref[...]


def _sem(e0, e1, e2, w1, b1, w2, wc1, bc1, wc2, bc2):
    return pl.pallas_call(
        _sem_body,
        grid=(GRID,),
        in_specs=[pl.BlockSpec((BLK, D), lambda i: (i, 0)),
                  pl.BlockSpec((BLK, D), lambda i: (i, 0)),
                  pl.BlockSpec((BLK, D), lambda i: (i, 0)),
                  pl.BlockSpec((D, D), lambda i: (0, 0)),
                  pl.BlockSpec((1, D), lambda i: (0, 0)),
                  pl.BlockSpec((1, D), lambda i: (0, 0)),
                  pl.BlockSpec((D, 64), lambda i: (0, 0)),
                  pl.BlockSpec((1, 64), lambda i: (0, 0)),
                  pl.BlockSpec((64, 2), lambda i: (0, 0)),
                  pl.BlockSpec((1, 2), lambda i: (0, 0))],
        out_specs=pl.BlockSpec((BLK, 2), lambda i: (i, 0)),
        out_shape=jax.ShapeDtypeStruct((NP, 2), jnp.float32),
    )(e0, e1, e2, w1, b1, w2, wc1, bc1, wc2, bc2)


def _edge_pass_body(src_hbm, dst_hbm, h_hbm, ta_hbm, tb_hbm, zo_hbm,
                    out_hbm, den_hbm,
                    sbuf, dbuf, csrc, cdst, sidx2, dloc2, dglob2,
                    hbuf, tabuf, tbbuf, sidx2b, dloc2b, dglob2b,
                    hbufb, tabufb, tbbufb, denblk, out_sh,
                    sem1, sem2, sem3, sem1b, sem2b, sem3b, semsc):
    cid = lax.axis_index("c")
    sid = lax.axis_index("s")
    wid = cid * NS + sid
    lane = lax.iota(jnp.int32, G)
    lmask = lane < NH
    loff = jnp.minimum(lane, NH - 1) * DRW
    zv = jnp.zeros((G,), jnp.float32)
    padv = jnp.full((G,), NP - 1, jnp.int32)

    def one_round(r, carry):
        base = r * RANGE
        pltpu.sync_copy(zo_hbm.at[pl.ds(0, RSTRIPE)],
                        out_sh.at[pl.ds(sid * RSTRIPE, RSTRIPE)])

        def zeroden(i, c):
            for k in range(8):
                denblk[pl.ds((i * 8 + k) * G, G)] = zv
            return c

        lax.fori_loop(0, NH * DRW // (8 * G), zeroden, 0)
        plsc.subcore_barrier()

        def scan_chunk(sc, cnt):
            ebase = wid * EPT + sc * SCC
            pltpu.sync_copy(src_hbm.at[pl.ds(ebase, SCC)], sbuf)
            pltpu.sync_copy(dst_hbm.at[pl.ds(ebase, SCC)], dbuf)

            def scan_group(g, cnt2):
                dv = dbuf[pl.ds(g * G, G)]
                sv = sbuf[pl.ds(g * G, G)]
                dl = dv - base
                m = (dl >= 0) & (dl < RANGE)
                plsc.store_compressed(csrc.at[pl.ds(cnt2, G)], sv, mask=m)
                plsc.store_compressed(cdst.at[pl.ds(cnt2, G)], dv, mask=m)
                return cnt2 + plsc.all_reduce_population_count(m)[0]

            return lax.fori_loop(0, SCC // G, scan_group, cnt)

        cnt = lax.fori_loop(0, NSC, scan_chunk, jnp.int32(0))
        for k in range(2 * C // G):
            csrc[pl.ds(cnt + k * G, G)] = padv
            cdst[pl.ds(cnt + k * G, G)] = padv
        nch2 = (cnt + 2 * C - 1) >> 7

        bufs = ((sidx2, dglob2, dloc2, hbuf, tabuf, tbbuf, sem1, sem2, sem3),
                (sidx2b, dglob2b, dloc2b, hbufb, tabufb, tbbufb,
                 sem1b, sem2b, sem3b))

        def prep_and_fire(j, b):
            si, dg, dl2, hb, tab, tbb, s1, s2, s3 = bufs[b]
            for k in range(NG):
                sv = csrc[pl.ds(j * C + k * G, G)]
                dv = cdst[pl.ds(j * C + k * G, G)]
                si[pl.ds(k * G, G)] = sv
                dg[pl.ds(k * G, G)] = dv
                dl = dv - base
                dl2[pl.ds(k * G, G)] = jnp.where((dl >= 0) & (dl < RANGE),
                                                 dl, RANGE)
            return (pltpu.async_copy(h_hbm.at[si], hb, s1),
                    pltpu.async_copy(ta_hbm.at[si], tab, s2),
                    pltpu.async_copy(tb_hbm.at[dg], tbb, s3))

        def compute(b):
            si, dg, dl2, hb, tab, tbb, s1, s2, s3 = bufs[b]

            def group(g, c2):
                dvec = dl2[pl.ds(g * G, G)]
                for j2 in range(G):
                    i = g * G + j2
                    e = tab[i, pl.ds(0, G)] + tbb[i, pl.ds(0, G)]
                    ex = jnp.exp(jnp.where(e >= 0.0, e, 0.2 * e))
                    plsc.addupdate_scatter(denblk, [dvec[j2] + loff], ex,
                                           mask=lmask)
                    for hh in range(NH):
                        s = ex[hh]
                        hb[i, pl.ds(hh * DH, DH)] = (
                            hb[i, pl.ds(hh * DH, DH)] * s)
                return c2

            lax.fori_loop(0, NG, group, 0)
            return pltpu.async_copy(hb, out_sh.at[dl2], add=True, sem=semsc)

        def step(jj, c1):
            cpa = prep_and_fire(jj * 2, 0)
            cpb = prep_and_fire(jj * 2 + 1, 1)
            for cp in cpa:
                cp.wait()
            sca = compute(0)
            sca.wait()
            for cp in cpb:
                cp.wait()
            scb = compute(1)
            scb.wait()
            return c1

        lax.fori_loop(0, nch2, step, 0)
        plsc.subcore_barrier()
        pltpu.sync_copy(out_sh.at[pl.ds(sid * RSTRIPE, RSTRIPE)],
                        out_hbm.at[cid, r, pl.ds(sid * RSTRIPE, RSTRIPE)])
        for hh in range(NH):
            pltpu.sync_copy(denblk.at[pl.ds(hh * DRW, RANGE)],
                            den_hbm.at[wid, hh, pl.ds(base, RANGE)])
        plsc.subcore_barrier()
        return carry

    lax.fori_loop(0, NR, one_round, 0)


@functools.cache
def _edge_pass():
    return pl.kernel(
        _edge_pass_body,
        out_type=(jax.ShapeDtypeStruct((NC, NR, RT, D), jnp.float32),
                  jax.ShapeDtypeStruct((NW, NH, NP), jnp.float32)),
        mesh=plsc.VectorSubcoreMesh(core_axis_name="c", subcore_axis_name="s",
                                    num_cores=NC, num_subcores=NS),
        compiler_params=pltpu.CompilerParams(needs_layout_passes=False),
        scratch_types=[
            pltpu.VMEM((SCC,), jnp.int32),
            pltpu.VMEM((SCC,), jnp.int32),
            pltpu.VMEM((EPT + 2 * C,), jnp.int32),
            pltpu.VMEM((EPT + 2 * C,), jnp.int32),
            pltpu.VMEM((C,), jnp.int32),
            pltpu.VMEM((C,), jnp.int32),
            pltpu.VMEM((C,), jnp.int32),
            pltpu.VMEM((C, D), jnp.float32),
            pltpu.VMEM((C, D), jnp.float32),
            pltpu.VMEM((C, D), jnp.float32),
            pltpu.VMEM((C,), jnp.int32),
            pltpu.VMEM((C,), jnp.int32),
            pltpu.VMEM((C,), jnp.int32),
            pltpu.VMEM((C, D), jnp.float32),
            pltpu.VMEM((C, D), jnp.float32),
            pltpu.VMEM((C, D), jnp.float32),
            pltpu.VMEM((NH * DRW,), jnp.float32),
            pltpu.VMEM_SHARED((RT, D), jnp.float32),
            pltpu.SemaphoreType.DMA,
            pltpu.SemaphoreType.DMA,
            pltpu.SemaphoreType.DMA,
            pltpu.SemaphoreType.DMA,
            pltpu.SemaphoreType.DMA,
            pltpu.SemaphoreType.DMA,
            pltpu.SemaphoreType.DMA,
        ],
    )


def _pad_edges(idx):
    idx = idx.astype(jnp.int32).reshape(NW, EPER)
    return jnp.pad(idx, ((0, 0), (0, EPT - EPER)), constant_values=N).reshape(-1)


def kernel(x_transaction, x_user, x_device,
           edge_index_tut, edge_index_tdt, edge_index_tudt,
           Wp_t, bp_t, Wp_u, bp_u, Wp_d, bp_d,
           Wgat, a_src, a_dst, b_gat,
           W_sem1, b_sem1, w_sem2, Wc1, bc1, Wc2, bc2):
    x = jnp.pad(x_transaction, ((0, NP - N), (0, 0)))
    h0 = _proj(x, Wp_t, bp_t.reshape(1, D))
    zo = jnp.zeros((STRIPE, D), jnp.float32)
    embs = []
    for p, ei in enumerate((edge_index_tut, edge_index_tdt, edge_index_tudt)):
        src = _pad_edges(ei[0])
        dst = _pad_edges(ei[1])
        hp = h0
        for l in range(2):
            ho, ta, tb, so, ds = _pre(hp, Wgat[p, l],
                                      a_src[p, l].reshape(1, D),
                                      a_dst[p, l].reshape(1, D))
            out2, den2 = _edge_pass()(src, dst, ho, ta, tb, zo)
            o0 = out2[0, :, :RANGE].reshape(NP, D)
            o1 = out2[1, :, :RANGE].reshape(NP, D)
            hp = _post(o0, o1, so, den2, ds,
                       b_gat[p, l].reshape(1, D))
        embs.append(hp)
    logits = _sem(embs[0], embs[1], embs[2],
                  W_sem1, b_sem1.reshape(1, D), w_sem2.reshape(1, D),
                  Wc1, bc1.reshape(1, 64), Wc2, bc2.reshape(1, 2))
    return logits[:N]


# speedup vs baseline: 1.0490x; 1.0490x over previous
import functools

import numpy as np
import jax
import jax.numpy as jnp
from jax import lax
from jax.experimental import pallas as pl
from jax.experimental.pallas import tpu as pltpu
from jax.experimental.pallas import tpu_sc as plsc

N = 10000
NP = 10240
D = 128
NH = 8
DH = 16
E = 320000
NC = 2
NS = 16
NW = NC * NS
EPER = E // NW
EPT = 10240
C = 128
G = 16
NG = C // G
NR = 10
RANGE = NP // NR
RT = 1152
RSTRIPE = RT // NS
DRW = RANGE + 128
SCC = 320
NSC = EPT // SCC
STRIPE = NP // NS
BLK = 1024
GRID = NP // BLK

_SEL = np.zeros((D, NH), np.float32)
for _d in range(D):
    _SEL[_d, _d // DH] = 1.0
_REP = np.zeros((2 * NH, D), np.float32)
for _h in range(NH):
    _REP[_h, _h * DH:(_h + 1) * DH] = 1.0
_REP8 = np.zeros((NH, D), np.float32)
for _h in range(NH):
    _REP8[_h, _h * DH:(_h + 1) * DH] = 1.0


def _proj_body(x_ref, w_ref, b_ref, o_ref):
    o_ref[...] = jnp.dot(x_ref[...], w_ref[...],
                         preferred_element_type=jnp.float32) + b_ref[...]


def _proj(x, w, b):
    k = x.shape[1]
    return pl.pallas_call(
        _proj_body,
        grid=(GRID,),
        in_specs=[pl.BlockSpec((BLK, k), lambda i: (i, 0)),
                  pl.BlockSpec((k, D), lambda i: (0, 0)),
                  pl.BlockSpec((1, D), lambda i: (0, 0))],
        out_specs=pl.BlockSpec((BLK, D), lambda i: (i, 0)),
        out_shape=jax.ShapeDtypeStruct((NP, D), jnp.float32),
    )(x, w, b)


def _pre_body(h_ref, w_ref, as_ref, ad_ref, sel_ref, rep_ref,
              ho_ref, ta_ref, tb_ref, so_ref, ds_ref):
    h = jnp.dot(h_ref[...], w_ref[...], preferred_element_type=jnp.float32)
    ho_ref[...] = h
    sel = sel_ref[...]
    als = jnp.dot(h * as_ref[...], sel, preferred_element_type=jnp.float32)
    ald = jnp.dot(h * ad_ref[...], sel, preferred_element_type=jnp.float32)
    ta_ref[...] = jnp.tile(als, (1, D // NH))
    tb_ref[...] = jnp.tile(ald, (1, D // NH))
    es = als + ald
    exs = jnp.exp(jnp.where(es >= 0.0, es, 0.2 * es))
    ds2 = jnp.concatenate([exs, exs], axis=1)
    ds_ref[...] = ds2
    so_ref[...] = h * jnp.dot(ds2, rep_ref[...],
                              preferred_element_type=jnp.float32)


def _pre(h, w, asf, adf):
    return pl.pallas_call(
        _pre_body,
        grid=(GRID,),
        in_specs=[pl.BlockSpec((BLK, D), lambda i: (i, 0)),
                  pl.BlockSpec((D, D), lambda i: (0, 0)),
                  pl.BlockSpec((1, D), lambda i: (0, 0)),
                  pl.BlockSpec((1, D), lambda i: (0, 0)),
                  pl.BlockSpec((D, NH), lambda i: (0, 0)),
                  pl.BlockSpec((2 * NH, D), lambda i: (0, 0))],
        out_specs=[pl.BlockSpec((BLK, D), lambda i: (i, 0)),
                   pl.BlockSpec((BLK, D), lambda i: (i, 0)),
                   pl.BlockSpec((BLK, D), lambda i: (i, 0)),
                   pl.BlockSpec((BLK, D), lambda i: (i, 0)),
                   pl.BlockSpec((BLK, 2 * NH), lambda i: (i, 0))],
        out_shape=[jax.ShapeDtypeStruct((NP, D), jnp.float32),
                   jax.ShapeDtypeStruct((NP, D), jnp.float32),
                   jax.ShapeDtypeStruct((NP, D), jnp.float32),
                   jax.ShapeDtypeStruct((NP, D), jnp.float32),
                   jax.ShapeDtypeStruct((NP, 2 * NH), jnp.float32)],
    )(h, w, asf, adf, _SEL, _REP)


def _post_body(o0_ref, o1_ref, so_ref, dsc_ref, ds_ref, b_ref, rep_ref,
               rep8_ref, ho_ref):
    dsum = jnp.sum(dsc_ref[...], axis=0)
    seg = lax.dot_general(dsum, rep8_ref[...], (((0,), (0,)), ((), ())),
                          preferred_element_type=jnp.float32)
    den = seg + jnp.dot(ds_ref[...], rep_ref[...],
                        preferred_element_type=jnp.float32)
    val = (o0_ref[...] + o1_ref[...] + so_ref[...]) / (den + 1e-16) + b_ref[...]
    act = jnp.where(val > 0.0, val, jnp.exp(val) - 1.0)
    rows = pl.program_id(0) * BLK + lax.broadcasted_iota(jnp.int32, (BLK, 1), 0)
    ho_ref[...] = jnp.where(rows < N, act, 0.0)


def _post(o0, o1, so, dsc, ds, b):
    return pl.pallas_call(
        _post_body,
        grid=(GRID,),
        in_specs=[pl.BlockSpec((BLK, D), lambda i: (i, 0)),
                  pl.BlockSpec((BLK, D), lambda i: (i, 0)),
                  pl.BlockSpec((BLK, D), lambda i: (i, 0)),
                  pl.BlockSpec((NW, NH, BLK), lambda i: (0, 0, i)),
                  pl.BlockSpec((BLK, 2 * NH), lambda i: (i, 0)),
                  pl.BlockSpec((1, D), lambda i: (0, 0)),
                  pl.BlockSpec((2 * NH, D), lambda i: (0, 0)),
                  pl.BlockSpec((NH, D), lambda i: (0, 0))],
        out_specs=pl.BlockSpec((BLK, D), lambda i: (i, 0)),
        out_shape=jax.ShapeDtypeStruct((NP, D), jnp.float32),
    )(o0, o1, so, dsc, ds, b, _REP, _REP8)


def _sem_body(e0_ref, e1_ref, e2_ref, w1_ref, b1_ref, w2_ref,
              wc1_ref, bc1_ref, wc2_ref, bc2_ref, o_ref):
    embs = [e0_ref[...], e1_ref[...], e2_ref[...]]
    w1 = w1_ref[...]
    b1 = b1_ref[...]
    w2 = w2_ref[...]
    ss = []
    for e in embs:
        t = jnp.tanh(jnp.dot(e, w1, preferred_element_type=jnp.float32) + b1)
        ss.append(jnp.sum(t * w2, axis=1, keepdims=True))
    m = jnp.maximum(jnp.maximum(ss[0], ss[1]), ss[2])
    ws = [jnp.exp(s - m) for s in ss]
    tot = ws[0] + ws[1] + ws[2]
    z = (ws[0] * embs[0] + ws[1] * embs[1] + ws[2] * embs[2]) / tot
    t1 = jnp.maximum(jnp.dot(z, wc1_ref[...],
                             preferred_element_type=jnp.float32) + bc1_ref[...],
                     0.0)
    o_ref[...] = jnp.dot(t1, wc2_ref[...],
                         preferred_element_type=jnp.float32) + bc2_ref[...]


def _sem(e0, e1, e2, w1, b1, w2, wc1, bc1, wc2, bc2):
    return pl.pallas_call(
        _sem_body,
        grid=(GRID,),
        in_specs=[pl.BlockSpec((BLK, D), lambda i: (i, 0)),
                  pl.BlockSpec((BLK, D), lambda i: (i, 0)),
                  pl.BlockSpec((BLK, D), lambda i: (i, 0)),
                  pl.BlockSpec((D, D), lambda i: (0, 0)),
                  pl.BlockSpec((1, D), lambda i: (0, 0)),
                  pl.BlockSpec((1, D), lambda i: (0, 0)),
                  pl.BlockSpec((D, 64), lambda i: (0, 0)),
                  pl.BlockSpec((1, 64), lambda i: (0, 0)),
                  pl.BlockSpec((64, 2), lambda i: (0, 0)),
                  pl.BlockSpec((1, 2), lambda i: (0, 0))],
        out_specs=pl.BlockSpec((BLK, 2), lambda i: (i, 0)),
        out_shape=jax.ShapeDtypeStruct((NP, 2), jnp.float32),
    )(e0, e1, e2, w1, b1, w2, wc1, bc1, wc2, bc2)


def _edge_pass_body(src_hbm, dst_hbm, h_hbm, ta_hbm, tb_hbm, zo_hbm,
                    out_hbm, den_hbm,
                    sbuf, dbuf, csrc, cdst, sidx2, dloc2, dglob2,
                    hbuf, tabuf, tbbuf, denblk, out_sh,
                    sem1, sem2, sem3):
    cid = lax.axis_index("c")
    sid = lax.axis_index("s")
    wid = cid * NS + sid
    lane = lax.iota(jnp.int32, G)
    lmask = lane < NH
    loff = jnp.minimum(lane, NH - 1) * DRW
    zv = jnp.zeros((G,), jnp.float32)
    padv = jnp.full((G,), NP - 1, jnp.int32)

    def one_round(r, carry):
        base = r * RANGE
        pltpu.sync_copy(zo_hbm.at[pl.ds(0, RSTRIPE)],
                        out_sh.at[pl.ds(sid * RSTRIPE, RSTRIPE)])

        def zeroden(i, c):
            for k in range(8):
                denblk[pl.ds((i * 8 + k) * G, G)] = zv
            return c

        lax.fori_loop(0, NH * DRW // (8 * G), zeroden, 0)
        plsc.subcore_barrier()

        def scan_chunk(sc, cnt):
            ebase = wid * EPT + sc * SCC
            pltpu.sync_copy(src_hbm.at[pl.ds(ebase, SCC)], sbuf)
            pltpu.sync_copy(dst_hbm.at[pl.ds(ebase, SCC)], dbuf)

            def scan_group(g, cnt2):
                dv = dbuf[pl.ds(g * G, G)]
                sv = sbuf[pl.ds(g * G, G)]
                dl = dv - base
                m = (dl >= 0) & (dl < RANGE)
                plsc.store_compressed(csrc.at[pl.ds(cnt2, G)], sv, mask=m)
                plsc.store_compressed(cdst.at[pl.ds(cnt2, G)], dv, mask=m)
                return cnt2 + plsc.all_reduce_population_count(m)[0]

            return lax.fori_loop(0, SCC // G, scan_group, cnt)

        cnt = lax.fori_loop(0, NSC, scan_chunk, jnp.int32(0))
        for k in range(C // G):
            csrc[pl.ds(cnt + k * G, G)] = padv
            cdst[pl.ds(cnt + k * G, G)] = padv
        nch = (cnt + C - 1) >> 7

        def chunk(j, c1):
            for k in range(NG):
                sv = csrc[pl.ds(j * C + k * G, G)]
                dv = cdst[pl.ds(j * C + k * G, G)]
                sidx2[pl.ds(k * G, G)] = sv
                dglob2[pl.ds(k * G, G)] = dv
                dl = dv - base
                dloc2[pl.ds(k * G, G)] = jnp.where((dl >= 0) & (dl < RANGE),
                                                   dl, RANGE)
            cp1 = pltpu.async_copy(h_hbm.at[sidx2], hbuf, sem1)
            cp2 = pltpu.async_copy(ta_hbm.at[sidx2], tabuf, sem2)
            cp3 = pltpu.async_copy(tb_hbm.at[dglob2], tbbuf, sem3)
            cp1.wait()
            cp2.wait()
            cp3.wait()

            def group(g, c2):
                dvec = dloc2[pl.ds(g * G, G)]
                for j2 in range(G):
                    i = g * G + j2
                    e = tabuf[i, pl.ds(0, G)] + tbbuf[i, pl.ds(0, G)]
                    ex = jnp.exp(jnp.where(e >= 0.0, e, 0.2 * e))
                    plsc.addupdate_scatter(denblk, [dvec[j2] + loff], ex,
                                           mask=lmask)
                    for hh in range(NH):
                        s = ex[hh]
                        hbuf[i, pl.ds(hh * DH, DH)] = (
                            hbuf[i, pl.ds(hh * DH, DH)] * s)
                return c2

            lax.fori_loop(0, NG, group, 0)
            pltpu.sync_copy(hbuf, out_sh.at[dloc2], add=True)
            return c1

        lax.fori_loop(0, nch, chunk, 0)
        plsc.subcore_barrier()
        pltpu.sync_copy(out_sh.at[pl.ds(sid * RSTRIPE, RSTRIPE)],
                        out_hbm.at[cid, r, pl.ds(sid * RSTRIPE, RSTRIPE)])
        for hh in range(NH):
            pltpu.sync_copy(denblk.at[pl.ds(hh * DRW, RANGE)],
                            den_hbm.at[wid, hh, pl.ds(base, RANGE)])
        plsc.subcore_barrier()
        return carry

    lax.fori_loop(0, NR, one_round, 0)


@functools.cache
def _edge_pass():
    return pl.kernel(
        _edge_pass_body,
        out_type=(jax.ShapeDtypeStruct((NC, NR, RT, D), jnp.float32),
                  jax.ShapeDtypeStruct((NW, NH, NP), jnp.float32)),
        mesh=plsc.VectorSubcoreMesh(core_axis_name="c", subcore_axis_name="s",
                                    num_cores=NC, num_subcores=NS),
        compiler_params=pltpu.CompilerParams(needs_layout_passes=False),
        scratch_types=[
            pltpu.VMEM((SCC,), jnp.int32),
            pltpu.VMEM((SCC,), jnp.int32),
            pltpu.VMEM((EPT + C,), jnp.int32),
            pltpu.VMEM((EPT + C,), jnp.int32),
            pltpu.VMEM((C,), jnp.int32),
            pltpu.VMEM((C,), jnp.int32),
            pltpu.VMEM((C,), jnp.int32),
            pltpu.VMEM((C, D), jnp.float32),
            pltpu.VMEM((C, D), jnp.float32),
            pltpu.VMEM((C, D), jnp.float32),
            pltpu.VMEM((NH * DRW,), jnp.float32),
            pltpu.VMEM_SHARED((RT, D), jnp.float32),
            pltpu.SemaphoreType.DMA,
            pltpu.SemaphoreType.DMA,
            pltpu.SemaphoreType.DMA,
        ],
    )


def _pad_edges(idx):
    idx = idx.astype(jnp.int32).reshape(NW, EPER)
    return jnp.pad(idx, ((0, 0), (0, EPT - EPER)), constant_values=N).reshape(-1)


def kernel(x_transaction, x_user, x_device,
           edge_index_tut, edge_index_tdt, edge_index_tudt,
           Wp_t, bp_t, Wp_u, bp_u, Wp_d, bp_d,
           Wgat, a_src, a_dst, b_gat,
           W_sem1, b_sem1, w_sem2, Wc1, bc1, Wc2, bc2):
    x = jnp.pad(x_transaction, ((0, NP - N), (0, 0)))
    h0 = _proj(x, Wp_t, bp_t.reshape(1, D))
    zo = jnp.zeros((STRIPE, D), jnp.float32)
    embs = []
    for p, ei in enumerate((edge_index_tut, edge_index_tdt, edge_index_tudt)):
        src = _pad_edges(ei[0])
        dst = _pad_edges(ei[1])
        hp = h0
        for l in range(2):
            ho, ta, tb, so, ds = _pre(hp, Wgat[p, l],
                                      a_src[p, l].reshape(1, D),
                                      a_dst[p, l].reshape(1, D))
            out2, den2 = _edge_pass()(src, dst, ho, ta, tb, zo)
            o0 = out2[0, :, :RANGE].reshape(NP, D)
            o1 = out2[1, :, :RANGE].reshape(NP, D)
            hp = _post(o0, o1, so, den2, ds,
                       b_gat[p, l].reshape(1, D))
        embs.append(hp)
    logits = _sem(embs[0], embs[1], embs[2],
                  W_sem1, b_sem1.reshape(1, D), w_sem2.reshape(1, D),
                  Wc1, bc1.reshape(1, 64), Wc2, bc2.reshape(1, 2))
    return logits[:N]


# speedup vs baseline: 1.3830x; 1.3184x over previous
import functools

import numpy as np
import jax
import jax.numpy as jnp
from jax import lax
from jax.experimental import pallas as pl
from jax.experimental.pallas import tpu as pltpu
from jax.experimental.pallas import tpu_sc as plsc

N = 10000
NP = 10240
D = 128
NH = 8
DH = 16
E = 320000
NC = 2
NS = 16
NW = NC * NS
EPER = E // NW
EPT = 10240
C = 32
G = 16
NG = C // G
NR = 10
RANGE = NP // NR
RT = 1152
RSTRIPE = RT // NS
DRW = RANGE + 128
SCC = 320
NSC = EPT // SCC
STRIPE = NP // NS
BLK = 1024
GRID = NP // BLK

_SEL = np.zeros((D, NH), np.float32)
for _d in range(D):
    _SEL[_d, _d // DH] = 1.0
_REP = np.zeros((2 * NH, D), np.float32)
for _h in range(NH):
    _REP[_h, _h * DH:(_h + 1) * DH] = 1.0
_REP8 = np.zeros((NH, D), np.float32)
for _h in range(NH):
    _REP8[_h, _h * DH:(_h + 1) * DH] = 1.0


def _proj_body(x_ref, w_ref, b_ref, o_ref):
    o_ref[...] = jnp.dot(x_ref[...], w_ref[...],
                         preferred_element_type=jnp.float32) + b_ref[...]


def _proj(x, w, b):
    k = x.shape[1]
    return pl.pallas_call(
        _proj_body,
        grid=(GRID,),
        in_specs=[pl.BlockSpec((BLK, k), lambda i: (i, 0)),
                  pl.BlockSpec((k, D), lambda i: (0, 0)),
                  pl.BlockSpec((1, D), lambda i: (0, 0))],
        out_specs=pl.BlockSpec((BLK, D), lambda i: (i, 0)),
        out_shape=jax.ShapeDtypeStruct((NP, D), jnp.float32),
    )(x, w, b)


def _pre_body(h_ref, w_ref, as_ref, ad_ref, sel_ref, rep_ref,
              ho_ref, ta_ref, tb_ref, so_ref, ds_ref):
    h = jnp.dot(h_ref[...], w_ref[...], preferred_element_type=jnp.float32)
    ho_ref[...] = h
    sel = sel_ref[...]
    als = jnp.dot(h * as_ref[...], sel, preferred_element_type=jnp.float32)
    ald = jnp.dot(h * ad_ref[...], sel, preferred_element_type=jnp.float32)
    ta_ref[...] = jnp.tile(als, (1, D // NH))
    tb_ref[...] = jnp.tile(ald, (1, D // NH))
    es = als + ald
    exs = jnp.exp(jnp.where(es >= 0.0, es, 0.2 * es))
    ds2 = jnp.concatenate([exs, exs], axis=1)
    ds_ref[...] = ds2
    so_ref[...] = h * jnp.dot(ds2, rep_ref[...],
                              preferred_element_type=jnp.float32)


def _pre(h, w, asf, adf):
    return pl.pallas_call(
        _pre_body,
        grid=(GRID,),
        in_specs=[pl.BlockSpec((BLK, D), lambda i: (i, 0)),
                  pl.BlockSpec((D, D), lambda i: (0, 0)),
                  pl.BlockSpec((1, D), lambda i: (0, 0)),
                  pl.BlockSpec((1, D), lambda i: (0, 0)),
                  pl.BlockSpec((D, NH), lambda i: (0, 0)),
                  pl.BlockSpec((2 * NH, D), lambda i: (0, 0))],
        out_specs=[pl.BlockSpec((BLK, D), lambda i: (i, 0)),
                   pl.BlockSpec((BLK, D), lambda i: (i, 0)),
                   pl.BlockSpec((BLK, D), lambda i: (i, 0)),
                   pl.BlockSpec((BLK, D), lambda i: (i, 0)),
                   pl.BlockSpec((BLK, 2 * NH), lambda i: (i, 0))],
        out_shape=[jax.ShapeDtypeStruct((NP, D), jnp.float32),
                   jax.ShapeDtypeStruct((NP, D), jnp.float32),
                   jax.ShapeDtypeStruct((NP, D), jnp.float32),
                   jax.ShapeDtypeStruct((NP, D), jnp.float32),
                   jax.ShapeDtypeStruct((NP, 2 * NH), jnp.float32)],
    )(h, w, asf, adf, _SEL, _REP)


def _post_body(o0_ref, o1_ref, so_ref, dsc_ref, ds_ref, b_ref, rep_ref,
               rep8_ref, ho_ref):
    dsum = jnp.sum(dsc_ref[...], axis=0)
    seg = lax.dot_general(dsum, rep8_ref[...], (((0,), (0,)), ((), ())),
                          preferred_element_type=jnp.float32)
    den = seg + jnp.dot(ds_ref[...], rep_ref[...],
                        preferred_element_type=jnp.float32)
    val = (o0_ref[...] + o1_ref[...] + so_ref[...]) / (den + 1e-16) + b_ref[...]
    act = jnp.where(val > 0.0, val, jnp.exp(val) - 1.0)
    rows = pl.program_id(0) * BLK + lax.broadcasted_iota(jnp.int32, (BLK, 1), 0)
    ho_ref[...] = jnp.where(rows < N, act, 0.0)


def _post(o0, o1, so, dsc, ds, b):
    return pl.pallas_call(
        _post_body,
        grid=(GRID,),
        in_specs=[pl.BlockSpec((BLK, D), lambda i: (i, 0)),
                  pl.BlockSpec((BLK, D), lambda i: (i, 0)),
                  pl.BlockSpec((BLK, D), lambda i: (i, 0)),
                  pl.BlockSpec((NW, NH, BLK), lambda i: (0, 0, i)),
                  pl.BlockSpec((BLK, 2 * NH), lambda i: (i, 0)),
                  pl.BlockSpec((1, D), lambda i: (0, 0)),
                  pl.BlockSpec((2 * NH, D), lambda i: (0, 0)),
                  pl.BlockSpec((NH, D), lambda i: (0, 0))],
        out_specs=pl.BlockSpec((BLK, D), lambda i: (i, 0)),
        out_shape=jax.ShapeDtypeStruct((NP, D), jnp.float32),
    )(o0, o1, so, dsc, ds, b, _REP, _REP8)


def _sem_body(e0_ref, e1_ref, e2_ref, w1_ref, b1_ref, w2_ref,
              wc1_ref, bc1_ref, wc2_ref, bc2_ref, o_ref):
    embs = [e0_ref[...], e1_ref[...], e2_ref[...]]
    w1 = w1_ref[...]
    b1 = b1_ref[...]
    w2 = w2_ref[...]
    ss = []
    for e in embs:
        t = jnp.tanh(jnp.dot(e, w1, preferred_element_type=jnp.float32) + b1)
        ss.append(jnp.sum(t * w2, axis=1, keepdims=True))
    m = jnp.maximum(jnp.maximum(ss[0], ss[1]), ss[2])
    ws = [jnp.exp(s - m) for s in ss]
    tot = ws[0] + ws[1] + ws[2]
    z = (ws[0] * embs[0] + ws[1] * embs[1] + ws[2] * embs[2]) / tot
    t1 = jnp.maximum(jnp.dot(z, wc1_ref[...],
                             preferred_element_type=jnp.float32) + bc1_ref[...],
                     0.0)
    o_ref[...] = jnp.dot(t1, wc2_ref[...],
                         preferred_element_type=jnp.float32) + bc2_ref[...]


def _sem(e0, e1, e2, w1, b1, w2, wc1, bc1, wc2, bc2):
    return pl.pallas_call(
        _sem_body,
        grid=(GRID,),
        in_specs=[pl.BlockSpec((BLK, D), lambda i: (i, 0)),
                  pl.BlockSpec((BLK, D), lambda i: (i, 0)),
                  pl.BlockSpec((BLK, D), lambda i: (i, 0)),
                  pl.BlockSpec((D, D), lambda i: (0, 0)),
                  pl.BlockSpec((1, D), lambda i: (0, 0)),
                  pl.BlockSpec((1, D), lambda i: (0, 0)),
                  pl.BlockSpec((D, 64), lambda i: (0, 0)),
                  pl.BlockSpec((1, 64), lambda i: (0, 0)),
                  pl.BlockSpec((64, 2), lambda i: (0, 0)),
                  pl.BlockSpec((1, 2), lambda i: (0, 0))],
        out_specs=pl.BlockSpec((BLK, 2), lambda i: (i, 0)),
        out_shape=jax.ShapeDtypeStruct((NP, 2), jnp.float32),
    )(e0, e1, e2, w1, b1, w2, wc1, bc1, wc2, bc2)


def _edge_pass_body(src_hbm, dst_hbm, h_hbm, ta_hbm, tb_hbm, zo_hbm,
                    out_hbm, den_hbm,
                    sbuf, dbuf, csrc, cdst, sidx2, dloc2, dglob2,
                    hbuf, tabuf, tbbuf, denblk, out_sh,
                    sem1, sem2, sem3):
    cid = lax.axis_index("c")
    sid = lax.axis_index("s")
    wid = cid * NS + sid
    lane = lax.iota(jnp.int32, G)
    lmask = lane < NH
    loff = jnp.minimum(lane, NH - 1) * DRW
    zv = jnp.zeros((G,), jnp.float32)
    padv = jnp.full((G,), NP - 1, jnp.int32)

    def one_round(r, carry):
        base = r * RANGE
        pltpu.sync_copy(zo_hbm.at[pl.ds(0, RSTRIPE)],
                        out_sh.at[pl.ds(sid * RSTRIPE, RSTRIPE)])

        def zeroden(i, c):
            for k in range(8):
                denblk[pl.ds((i * 8 + k) * G, G)] = zv
            return c

        lax.fori_loop(0, NH * DRW // (8 * G), zeroden, 0)
        plsc.subcore_barrier()

        def scan_chunk(sc, cnt):
            ebase = wid * EPT + sc * SCC
            pltpu.sync_copy(src_hbm.at[pl.ds(ebase, SCC)], sbuf)
            pltpu.sync_copy(dst_hbm.at[pl.ds(ebase, SCC)], dbuf)

            def scan_group(g, cnt2):
                dv = dbuf[pl.ds(g * G, G)]
                sv = sbuf[pl.ds(g * G, G)]
                dl = dv - base
                m = (dl >= 0) & (dl < RANGE)
                plsc.store_compressed(csrc.at[pl.ds(cnt2, G)], sv, mask=m)
                plsc.store_compressed(cdst.at[pl.ds(cnt2, G)], dv, mask=m)
                return cnt2 + plsc.all_reduce_population_count(m)[0]

            return lax.fori_loop(0, SCC // G, scan_group, cnt)

        cnt = lax.fori_loop(0, NSC, scan_chunk, jnp.int32(0))
        for k in range(C // G):
            csrc[pl.ds(cnt + k * G, G)] = padv
            cdst[pl.ds(cnt + k * G, G)] = padv
        nch = (cnt + C - 1) >> 5

        def chunk(j, c1):
            for k in range(NG):
                sv = csrc[pl.ds(j * C + k * G, G)]
                dv = cdst[pl.ds(j * C + k * G, G)]
                sidx2[pl.ds(k * G, G)] = sv
                dglob2[pl.ds(k * G, G)] = dv
                dl = dv - base
                dloc2[pl.ds(k * G, G)] = jnp.where((dl >= 0) & (dl < RANGE),
                                                   dl, RANGE)
            cp1 = pltpu.async_copy(h_hbm.at[sidx2], hbuf, sem1)
            cp2 = pltpu.async_copy(ta_hbm.at[sidx2], tabuf, sem2)
            cp3 = pltpu.async_copy(tb_hbm.at[dglob2], tbbuf, sem3)
            cp1.wait()
            cp2.wait()
            cp3.wait()

            def group(g, c2):
                dvec = dloc2[pl.ds(g * G, G)]
                for j2 in range(G):
                    i = g * G + j2
                    e = tabuf[i, pl.ds(0, G)] + tbbuf[i, pl.ds(0, G)]
                    ex = jnp.exp(jnp.where(e >= 0.0, e, 0.2 * e))
                    plsc.addupdate_scatter(denblk, [dvec[j2] + loff], ex,
                                           mask=lmask)
                    for hh in range(NH):
                        s = ex[hh]
                        hbuf[i, pl.ds(hh * DH, DH)] = (
                            hbuf[i, pl.ds(hh * DH, DH)] * s)
                return c2

            lax.fori_loop(0, NG, group, 0)
            pltpu.sync_copy(hbuf, out_sh.at[dloc2], add=True)
            return c1

        lax.fori_loop(0, nch, chunk, 0)
        plsc.subcore_barrier()
        pltpu.sync_copy(out_sh.at[pl.ds(sid * RSTRIPE, RSTRIPE)],
                        out_hbm.at[cid, r, pl.ds(sid * RSTRIPE, RSTRIPE)])
        for hh in range(NH):
            pltpu.sync_copy(denblk.at[pl.ds(hh * DRW, RANGE)],
                            den_hbm.at[wid, hh, pl.ds(base, RANGE)])
        plsc.subcore_barrier()
        return carry

    lax.fori_loop(0, NR, one_round, 0)


@functools.cache
def _edge_pass():
    return pl.kernel(
        _edge_pass_body,
        out_type=(jax.ShapeDtypeStruct((NC, NR, RT, D), jnp.float32),
                  jax.ShapeDtypeStruct((NW, NH, NP), jnp.float32)),
        mesh=plsc.VectorSubcoreMesh(core_axis_name="c", subcore_axis_name="s",
                                    num_cores=NC, num_subcores=NS),
        compiler_params=pltpu.CompilerParams(needs_layout_passes=False),
        scratch_types=[
            pltpu.VMEM((SCC,), jnp.int32),
            pltpu.VMEM((SCC,), jnp.int32),
            pltpu.VMEM((EPT + C,), jnp.int32),
            pltpu.VMEM((EPT + C,), jnp.int32),
            pltpu.VMEM((C,), jnp.int32),
            pltpu.VMEM((C,), jnp.int32),
            pltpu.VMEM((C,), jnp.int32),
            pltpu.VMEM((C, D), jnp.float32),
            pltpu.VMEM((C, D), jnp.float32),
            pltpu.VMEM((C, D), jnp.float32),
            pltpu.VMEM((NH * DRW,), jnp.float32),
            pltpu.VMEM_SHARED((RT, D), jnp.float32),
            pltpu.SemaphoreType.DMA,
            pltpu.SemaphoreType.DMA,
            pltpu.SemaphoreType.DMA,
        ],
    )


def _pad_edges(idx):
    idx = idx.astype(jnp.int32).reshape(NW, EPER)
    return jnp.pad(idx, ((0, 0), (0, EPT - EPER)), constant_values=N).reshape(-1)


def kernel(x_transaction, x_user, x_device,
           edge_index_tut, edge_index_tdt, edge_index_tudt,
           Wp_t, bp_t, Wp_u, bp_u, Wp_d, bp_d,
           Wgat, a_src, a_dst, b_gat,
           W_sem1, b_sem1, w_sem2, Wc1, bc1, Wc2, bc2):
    x = jnp.pad(x_transaction, ((0, NP - N), (0, 0)))
    h0 = _proj(x, Wp_t, bp_t.reshape(1, D))
    zo = jnp.zeros((STRIPE, D), jnp.float32)
    embs = []
    for p, ei in enumerate((edge_index_tut, edge_index_tdt, edge_index_tudt)):
        src = _pad_edges(ei[0])
        dst = _pad_edges(ei[1])
        hp = h0
        for l in range(2):
            ho, ta, tb, so, ds = _pre(hp, Wgat[p, l],
                                      a_src[p, l].reshape(1, D),
                                      a_dst[p, l].reshape(1, D))
            out2, den2 = _edge_pass()(src, dst, ho, ta, tb, zo)
            o0 = out2[0, :, :RANGE].reshape(NP, D)
            o1 = out2[1, :, :RANGE].reshape(NP, D)
            hp = _post(o0, o1, so, den2, ds,
                       b_gat[p, l].reshape(1, D))
        embs.append(hp)
    logits = _sem(embs[0], embs[1], embs[2],
                  W_sem1, b_sem1.reshape(1, D), w_sem2.reshape(1, D),
                  Wc1, bc1.reshape(1, 64), Wc2, bc2.reshape(1, 2))
    return logits[:N]
